# Initial kernel scaffold; baseline (speedup 1.0000x reference)
import jax
import jax.numpy as jnp
from jax.experimental import pallas as pl


def kernel(adj, features, nodes_batch, W0, W1):
    raise NotImplementedError("write your pallas kernel here")


# speedup vs baseline: 4.7954x; 4.7954x over previous
import functools

import jax
import jax.numpy as jnp
from jax import lax
from jax.experimental import pallas as pl
from jax.experimental.pallas import tpu as pltpu
from jax.experimental.pallas import tpu_sc as plsc

NC = 2
NS = 16
NW = NC * NS


def _mm_kernel(x_ref, w_ref, o_ref):
    o_ref[...] = jnp.dot(x_ref[...], w_ref[...],
                         preferred_element_type=jnp.float32)


def _tc_matmul(x, w, bm):
    m, k = x.shape
    n = w.shape[1]
    grid = (m // bm,)
    return pl.pallas_call(
        _mm_kernel,
        grid=grid,
        in_specs=[
            pl.BlockSpec((bm, k), lambda i: (i, 0)),
            pl.BlockSpec((k, n), lambda i: (0, 0)),
        ],
        out_specs=pl.BlockSpec((bm, n), lambda i: (i, 0)),
        out_shape=jax.ShapeDtypeStruct((m, n), jnp.float32),
    )(x, w)


def _mm2_kernel(a_ref, b_ref, w_ref, o_ref):
    o_ref[...] = jnp.dot(a_ref[...] + b_ref[...], w_ref[...],
                         preferred_element_type=jnp.float32)


def _tc_add_matmul(a, b, w, bm):
    m, k = a.shape
    n = w.shape[1]
    grid = (m // bm,)
    return pl.pallas_call(
        _mm2_kernel,
        grid=grid,
        in_specs=[
            pl.BlockSpec((bm, k), lambda i: (i, 0)),
            pl.BlockSpec((bm, k), lambda i: (i, 0)),
            pl.BlockSpec((k, n), lambda i: (0, 0)),
        ],
        out_specs=pl.BlockSpec((bm, n), lambda i: (i, 0)),
        out_shape=jax.ShapeDtypeStruct((m, n), jnp.float32),
    )(a, b, w)


def _zzt_kernel(z_blk_ref, z_all_ref, o_ref):
    o_ref[...] = lax.dot_general(
        z_blk_ref[...], z_all_ref[...],
        dimension_numbers=(((1,), (1,)), ((), ())),
        preferred_element_type=jnp.float32)


def _tc_zzt(z, bm):
    nb, d = z.shape
    grid = (nb // bm,)
    return pl.pallas_call(
        _zzt_kernel,
        grid=grid,
        in_specs=[
            pl.BlockSpec((bm, d), lambda i: (i, 0)),
            pl.BlockSpec((nb, d), lambda i: (0, 0)),
        ],
        out_specs=pl.BlockSpec((bm, nb), lambda i: (i, 0)),
        out_shape=jax.ShapeDtypeStruct((nb, nb), jnp.float32),
    )(z, z)


def _sc_segment_sum(h, src, dst, zeros, n_pad, chunk):
    e = src.shape[0]
    d = h.shape[1]
    assert e % NW == 0
    e_per_w = e // NW
    assert e_per_w % chunk == 0 and chunk % 8 == 0 and chunk <= 128
    n_ch = e_per_w // chunk
    assert n_pad % (NS * 8) == 0
    rows_per_tile = n_pad // NS

    mesh = plsc.VectorSubcoreMesh(core_axis_name="c", subcore_axis_name="s")

    @functools.partial(
        pl.kernel,
        out_type=jax.ShapeDtypeStruct((NC, n_pad, d), jnp.float32),
        mesh=mesh,
        scratch_types=[
            pltpu.VMEM((chunk,), jnp.int32),
            pltpu.VMEM((chunk,), jnp.int32),
            pltpu.VMEM((chunk, d), jnp.float32),
            pltpu.VMEM_SHARED((n_pad, d), jnp.float32),
            pltpu.SemaphoreType.DMA,
        ],
    )
    def seg(h_hbm, src_hbm, dst_hbm, z_hbm, out_hbm, sidx, didx, rows, acc,
            sem):
        c = lax.axis_index("c")
        s = lax.axis_index("s")
        wid = c * NS + s
        row0 = pl.multiple_of(s * rows_per_tile, 8)
        pltpu.sync_copy(z_hbm.at[pl.ds(row0, rows_per_tile)],
                        acc.at[pl.ds(row0, rows_per_tile)])
        plsc.subcore_barrier()

        base = wid * e_per_w

        def body(j, carry):
            off = pl.multiple_of(base + j * chunk, 8)
            pltpu.sync_copy(src_hbm.at[pl.ds(off, chunk)], sidx)
            pltpu.sync_copy(dst_hbm.at[pl.ds(off, chunk)], didx)
            pltpu.async_copy(h_hbm.at[sidx], rows, sem).wait()
            pltpu.sync_copy(rows, acc.at[didx], add=True)
            return carry

        lax.fori_loop(0, n_ch, body, 0)
        plsc.subcore_barrier()
        pltpu.sync_copy(acc.at[pl.ds(row0, rows_per_tile)],
                        out_hbm.at[c].at[pl.ds(row0, rows_per_tile)])

    return seg(h, src, dst, zeros)


def _sc_gather_add_relu(q0, q1, nb):
    b = nb.shape[0]
    d = q0.shape[1]
    assert b % NW == 0
    b_per_w = b // NW
    assert b_per_w % 8 == 0 and b_per_w <= 128 and d % 16 == 0

    mesh = plsc.VectorSubcoreMesh(core_axis_name="c", subcore_axis_name="s")

    @functools.partial(
        pl.kernel,
        out_type=jax.ShapeDtypeStruct((b, d), jnp.float32),
        mesh=mesh,
        scratch_types=[
            pltpu.VMEM((b_per_w,), jnp.int32),
            pltpu.VMEM((b_per_w, d), jnp.float32),
            pltpu.VMEM((b_per_w, d), jnp.float32),
            pltpu.SemaphoreType.DMA,
        ],
    )
    def gat(q0_hbm, q1_hbm, nb_hbm, out_hbm, idx, b0, b1, sem):
        c = lax.axis_index("c")
        s = lax.axis_index("s")
        wid = c * NS + s
        base = pl.multiple_of(wid * b_per_w, 8)
        pltpu.sync_copy(nb_hbm.at[pl.ds(base, b_per_w)], idx)
        pltpu.async_copy(q0_hbm.at[idx], b0, sem).wait()
        pltpu.async_copy(q1_hbm.at[idx], b1, sem).wait()

        def body(i, carry):
            for j in range(d // 16):
                sl = pl.ds(j * 16, 16)
                v = b0[i, sl] + b1[i, sl]
                b0[i, sl] = jnp.maximum(v, 0.0)
            return carry

        lax.fori_loop(0, b_per_w, body, 0)
        pltpu.sync_copy(b0, out_hbm.at[pl.ds(base, b_per_w)])

    return gat(q0, q1, nb)


def kernel(adj, features, nodes_batch, W0, W1):
    n_nodes = features.shape[0]
    n_pad = ((n_nodes + NS * 8 - 1) // (NS * 8)) * (NS * 8)
    src = adj[0].astype(jnp.int32)
    dst = adj[1].astype(jnp.int32)
    nb = nodes_batch.astype(jnp.int32)

    hidden_dim = W0.shape[1]
    emb = W1.shape[1]
    w1p = jnp.pad(W1, ((0, 0), (0, hidden_dim - emb)))

    zeros_h = jnp.zeros((n_pad, hidden_dim), jnp.float32)

    h0 = _tc_matmul(features, W0, bm=1000)
    p = _sc_segment_sum(h0, src, dst, zeros_h, n_pad, chunk=80)
    h1 = _tc_add_matmul(p[0], p[1], w1p, bm=n_pad // NS)
    q = _sc_segment_sum(h1, src, dst, zeros_h, n_pad, chunk=80)
    z = _sc_gather_add_relu(q[0], q[1], nb)
    return _tc_zzt(z, bm=256)


# speedup vs baseline: 6.6715x; 1.3912x over previous
import functools

import jax
import jax.numpy as jnp
from jax import lax
from jax.experimental import pallas as pl
from jax.experimental.pallas import tpu as pltpu
from jax.experimental.pallas import tpu_sc as plsc

NC = 2
NS = 16
NW = NC * NS
NSLOT = 5


def _mm_kernel(x_ref, w_ref, o_ref):
    o_ref[...] = jnp.dot(x_ref[...], w_ref[...],
                         preferred_element_type=jnp.float32)


def _tc_matmul(x, w, bm):
    m, k = x.shape
    n = w.shape[1]
    grid = (m // bm,)
    return pl.pallas_call(
        _mm_kernel,
        grid=grid,
        in_specs=[
            pl.BlockSpec((bm, k), lambda i: (i, 0)),
            pl.BlockSpec((k, n), lambda i: (0, 0)),
        ],
        out_specs=pl.BlockSpec((bm, n), lambda i: (i, 0)),
        out_shape=jax.ShapeDtypeStruct((m, n), jnp.float32),
    )(x, w)


def _mm2_kernel(a_ref, b_ref, w_ref, o_ref):
    o_ref[...] = jnp.dot(a_ref[...] + b_ref[...], w_ref[...],
                         preferred_element_type=jnp.float32)


def _tc_add_matmul(a, b, w, bm):
    m, k = a.shape
    n = w.shape[1]
    grid = (m // bm,)
    return pl.pallas_call(
        _mm2_kernel,
        grid=grid,
        in_specs=[
            pl.BlockSpec((bm, k), lambda i: (i, 0)),
            pl.BlockSpec((bm, k), lambda i: (i, 0)),
            pl.BlockSpec((k, n), lambda i: (0, 0)),
        ],
        out_specs=pl.BlockSpec((bm, n), lambda i: (i, 0)),
        out_shape=jax.ShapeDtypeStruct((m, n), jnp.float32),
    )(a, b, w)


def _zzt_kernel(z_blk_ref, z_all_ref, o_ref):
    o_ref[...] = lax.dot_general(
        z_blk_ref[...], z_all_ref[...],
        dimension_numbers=(((1,), (1,)), ((), ())),
        preferred_element_type=jnp.float32)


def _tc_zzt(z, bm):
    nb, d = z.shape
    grid = (nb // bm,)
    return pl.pallas_call(
        _zzt_kernel,
        grid=grid,
        in_specs=[
            pl.BlockSpec((bm, d), lambda i: (i, 0)),
            pl.BlockSpec((nb, d), lambda i: (0, 0)),
        ],
        out_specs=pl.BlockSpec((bm, nb), lambda i: (i, 0)),
        out_shape=jax.ShapeDtypeStruct((nb, nb), jnp.float32),
    )(z, z)


def _sc_segment_sum(h, src3, dst3, zeros, n_pad, chunk):
    _, n_ch, _ = src3.shape
    d = h.shape[1]
    assert src3.shape == dst3.shape == (NW, n_ch, chunk)
    assert chunk % 8 == 0 and chunk <= 128
    assert n_pad % (NS * 8) == 0
    rows_per_tile = n_pad // NS

    mesh = plsc.VectorSubcoreMesh(core_axis_name="c", subcore_axis_name="s")

    @functools.partial(
        pl.kernel,
        out_type=jax.ShapeDtypeStruct((NC, n_pad, d), jnp.float32),
        mesh=mesh,
        scratch_types=[
            pltpu.VMEM((n_ch, chunk), jnp.int32),
            pltpu.VMEM((n_ch, chunk), jnp.int32),
            pltpu.VMEM((chunk, d), jnp.float32),
            pltpu.VMEM_SHARED((n_pad, d), jnp.float32),
            pltpu.SemaphoreType.DMA,
            pltpu.SemaphoreType.DMA,
        ],
    )
    def seg(h_hbm, src_hbm, dst_hbm, z_hbm, out_hbm, sidx, didx, rows, acc,
            sem, isem):
        c = lax.axis_index("c")
        s = lax.axis_index("s")
        wid = c * NS + s
        row0 = pl.multiple_of(s * rows_per_tile, 8)
        pltpu.async_copy(src_hbm.at[wid], sidx, isem)
        pltpu.async_copy(dst_hbm.at[wid], didx, isem)
        pltpu.sync_copy(z_hbm, acc.at[pl.ds(row0, rows_per_tile)])
        pltpu.make_async_copy(src_hbm.at[wid], sidx, isem).wait()
        pltpu.make_async_copy(dst_hbm.at[wid], didx, isem).wait()
        plsc.subcore_barrier()

        def body(j, carry):
            pltpu.async_copy(h_hbm.at[sidx.at[j]], rows, sem).wait()
            pltpu.sync_copy(rows, acc.at[didx.at[j]], add=True)
            return carry

        lax.fori_loop(0, n_ch, body, 0)
        plsc.subcore_barrier()
        pltpu.sync_copy(acc.at[pl.ds(row0, rows_per_tile)],
                        out_hbm.at[c].at[pl.ds(row0, rows_per_tile)])

    return seg(h, src3, dst3, zeros)


def _sc_gather_add_relu(q0, q1, nb):
    b = nb.shape[0]
    d = q0.shape[1]
    assert b % NW == 0
    b_per_w = b // NW
    assert b_per_w % 8 == 0 and b_per_w <= 128 and d % 16 == 0

    mesh = plsc.VectorSubcoreMesh(core_axis_name="c", subcore_axis_name="s")

    @functools.partial(
        pl.kernel,
        out_type=jax.ShapeDtypeStruct((b, d), jnp.float32),
        mesh=mesh,
        scratch_types=[
            pltpu.VMEM((b_per_w,), jnp.int32),
            pltpu.VMEM((b_per_w, d), jnp.float32),
            pltpu.VMEM((b_per_w, d), jnp.float32),
            pltpu.SemaphoreType.DMA,
        ],
    )
    def gat(q0_hbm, q1_hbm, nb_hbm, out_hbm, idx, b0, b1, sem):
        c = lax.axis_index("c")
        s = lax.axis_index("s")
        wid = c * NS + s
        base = pl.multiple_of(wid * b_per_w, 8)
        pltpu.sync_copy(nb_hbm.at[pl.ds(base, b_per_w)], idx)
        pltpu.async_copy(q0_hbm.at[idx], b0, sem).wait()
        pltpu.async_copy(q1_hbm.at[idx], b1, sem).wait()

        def body(i, carry):
            for j in range(d // 16):
                sl = pl.ds(j * 16, 16)
                v = b0[i, sl] + b1[i, sl]
                b0[i, sl] = jnp.maximum(v, 0.0)
            return carry

        lax.fori_loop(0, b_per_w, body, 0)
        pltpu.sync_copy(b0, out_hbm.at[pl.ds(base, b_per_w)])

    return gat(q0, q1, nb)


def kernel(adj, features, nodes_batch, W0, W1):
    n_nodes = features.shape[0]
    n_pad = ((n_nodes + NS * 8 - 1) // (NS * 8)) * (NS * 8)
    chunk = 80
    e = adj.shape[1]
    n_ch = e // (NW * chunk)
    src3 = adj[0].astype(jnp.int32).reshape(NW, n_ch, chunk)
    dst3 = adj[1].astype(jnp.int32).reshape(NW, n_ch, chunk)
    nb = nodes_batch.astype(jnp.int32)

    hidden_dim = W0.shape[1]
    emb = W1.shape[1]
    w1p = jnp.pad(W1, ((0, 0), (0, hidden_dim - emb)))

    zeros_h = jnp.zeros((n_pad // NS, hidden_dim), jnp.float32)

    fpad = jnp.pad(features, ((0, n_pad - n_nodes), (0, 0)))
    h0 = _tc_matmul(fpad, W0, bm=n_pad // NS)
    p = _sc_segment_sum(h0, src3, dst3, zeros_h, n_pad, chunk)
    h1 = _tc_add_matmul(p[0], p[1], w1p, bm=n_pad // NS)
    q = _sc_segment_sum(h1, src3, dst3, zeros_h, n_pad, chunk)
    z = _sc_gather_add_relu(q[0], q[1], nb)
    return _tc_zzt(z, bm=256)
